# Initial kernel scaffold; baseline (speedup 1.0000x reference)
import jax
import jax.numpy as jnp
from jax.experimental import pallas as pl


def kernel(x, cls_labels, reg_labels, cls_masks, reg_masks):
    raise NotImplementedError("write your pallas kernel here")


# speedup vs baseline: 2.3316x; 2.3316x over previous
import jax
import jax.numpy as jnp
import numpy as np
from jax.experimental import pallas as pl
from jax.experimental.pallas import tpu as pltpu

_MIX_BETA = 0.5
_BS = 64


def _mix_consts():
    key = jax.random.key(42)
    k_perm, k_beta = jax.random.split(key)
    perm = jax.random.permutation(k_perm, _BS)
    coeffs = jax.random.beta(k_beta, _MIX_BETA, _MIX_BETA, shape=(_BS,)).astype(
        jnp.float32
    )
    return np.asarray(perm), np.asarray(coeffs)


_PERM_NP, _COEFFS_NP = _mix_consts()


def _mix_body(idx_ref, c_ref, x_ref, xp_ref, cls_ref, reg_ref, cm_ref, rm_ref,
              out_x, o_cls1, o_cls2, o_reg1, o_reg2, o_cm1, o_cm2, o_rm1, o_rm2):
    i = pl.program_id(0)
    c = c_ref[i]
    out_x[...] = c * x_ref[...] + (1.0 - c) * xp_ref[...]

    @pl.when(i == 0)
    def _():
        o_cls1[...] = cls_ref[...]
        o_reg1[...] = reg_ref[...]
        o_cm1[...] = cm_ref[...]
        o_rm1[...] = rm_ref[...]

    j = idx_ref[i]
    o_cls2[pl.ds(i, 1), :] = cls_ref[pl.ds(j, 1), :]
    o_reg2[pl.ds(i, 1), :] = reg_ref[pl.ds(j, 1), :]
    o_cm2[pl.ds(i, 1), :] = cm_ref[pl.ds(j, 1), :]
    o_rm2[pl.ds(i, 1), :] = rm_ref[pl.ds(j, 1), :]


def kernel(x, cls_labels, reg_labels, cls_masks, reg_masks):
    bs = x.shape[0]
    lab_shape = cls_labels.shape
    xblock = (1,) + x.shape[1:]

    idx = jnp.asarray(_PERM_NP, dtype=jnp.int32)
    coeffs = jnp.asarray(_COEFFS_NP, dtype=jnp.float32)

    xspec = pl.BlockSpec(xblock, lambda i, idx_ref, c_ref: (i, 0, 0, 0))
    xpspec = pl.BlockSpec(xblock, lambda i, idx_ref, c_ref: (idx_ref[i], 0, 0, 0))
    lspec = pl.BlockSpec(lab_shape, lambda i, idx_ref, c_ref: (0, 0))

    grid_spec = pltpu.PrefetchScalarGridSpec(
        num_scalar_prefetch=2,
        grid=(bs,),
        in_specs=[xspec, xpspec, lspec, lspec, lspec, lspec],
        out_specs=[xspec] + [lspec] * 8,
    )
    lab_sds = jax.ShapeDtypeStruct(lab_shape, cls_labels.dtype)
    outs = pl.pallas_call(
        _mix_body,
        grid_spec=grid_spec,
        out_shape=[jax.ShapeDtypeStruct(x.shape, x.dtype)] + [lab_sds] * 8,
    )(idx, coeffs, x, x, cls_labels, reg_labels, cls_masks, reg_masks)
    (x_mixed, cls1, cls2, reg1, reg2, cm1, cm2, rm1, rm2) = outs
    return (x_mixed, cls1, cls2, reg1, reg2, cm1, cm2, rm1, rm2, coeffs, idx)


# speedup vs baseline: 2.5133x; 1.0780x over previous
import jax
import jax.numpy as jnp
import numpy as np
from jax.experimental import pallas as pl
from jax.experimental.pallas import tpu as pltpu

_MIX_BETA = 0.5
_BS = 64


def _mix_consts():
    key = jax.random.key(42)
    k_perm, k_beta = jax.random.split(key)
    perm = jax.random.permutation(k_perm, _BS)
    coeffs = jax.random.beta(k_beta, _MIX_BETA, _MIX_BETA, shape=(_BS,)).astype(
        jnp.float32
    )
    return np.asarray(perm), np.asarray(coeffs)


_PERM_NP, _COEFFS_NP = _mix_consts()


def _cycle_schedule(perm, coeffs):
    n = len(perm)
    seen = np.zeros(n, dtype=bool)
    load_idx, out_idx, flags, c_step = [], [], [], []
    for s in range(n):
        if seen[s]:
            continue
        cyc = []
        j = s
        while not seen[j]:
            seen[j] = True
            cyc.append(j)
            j = int(perm[j])
        load_idx.append(cyc[0])
        out_idx.append(cyc[0])
        flags.append(1)
        c_step.append(coeffs[cyc[0]])
        for k in range(1, len(cyc)):
            load_idx.append(cyc[k])
            out_idx.append(cyc[k - 1])
            flags.append(0)
            c_step.append(coeffs[cyc[k - 1]])
        load_idx.append(cyc[-1])
        out_idx.append(cyc[-1])
        flags.append(2)
        c_step.append(coeffs[cyc[-1]])
    return (
        np.asarray(load_idx, np.int32),
        np.asarray(out_idx, np.int32),
        np.asarray(flags, np.int32),
        np.asarray(c_step, np.float32),
    )


_LOAD_NP, _OUT_NP, _FLAGS_NP, _CSTEP_NP = _cycle_schedule(_PERM_NP, _COEFFS_NP)
_NSTEPS = len(_LOAD_NP)


def _mix_body(pidx_ref, lidx_ref, oidx_ref, flags_ref, c_ref,
              x_ref, cls_ref, reg_ref, cm_ref, rm_ref,
              out_x, o_cls1, o_cls2, o_reg1, o_reg2, o_cm1, o_cm2, o_rm1, o_rm2,
              prev, headbuf):
    g = pl.program_id(0)
    f = flags_ref[g]
    c = c_ref[g]

    @pl.when(f == 1)
    def _():
        prev[...] = x_ref[...]
        headbuf[...] = x_ref[...]

    @pl.when(f == 0)
    def _():
        out_x[...] = c * prev[...] + (1.0 - c) * x_ref[...]
        prev[...] = x_ref[...]

    @pl.when(f == 2)
    def _():
        out_x[...] = c * prev[...] + (1.0 - c) * headbuf[...]

    @pl.when(g == 0)
    def _():
        o_cls1[...] = cls_ref[...]
        o_reg1[...] = reg_ref[...]
        o_cm1[...] = cm_ref[...]
        o_rm1[...] = rm_ref[...]

    @pl.when(g < _BS)
    def _():
        j = pidx_ref[g]
        o_cls2[pl.ds(g, 1), :] = cls_ref[pl.ds(j, 1), :]
        o_reg2[pl.ds(g, 1), :] = reg_ref[pl.ds(j, 1), :]
        o_cm2[pl.ds(g, 1), :] = cm_ref[pl.ds(j, 1), :]
        o_rm2[pl.ds(g, 1), :] = rm_ref[pl.ds(j, 1), :]


def kernel(x, cls_labels, reg_labels, cls_masks, reg_masks):
    lab_shape = cls_labels.shape
    xblock = (1,) + x.shape[1:]

    pidx = jnp.asarray(_PERM_NP, dtype=jnp.int32)
    coeffs = jnp.asarray(_COEFFS_NP, dtype=jnp.float32)
    lidx = jnp.asarray(_LOAD_NP)
    oidx = jnp.asarray(_OUT_NP)
    flags = jnp.asarray(_FLAGS_NP)
    cstep = jnp.asarray(_CSTEP_NP)

    xspec = pl.BlockSpec(xblock, lambda g, p, l, o, f, c: (l[g], 0, 0, 0))
    ospec = pl.BlockSpec(xblock, lambda g, p, l, o, f, c: (o[g], 0, 0, 0))
    lspec = pl.BlockSpec(lab_shape, lambda g, p, l, o, f, c: (0, 0))

    grid_spec = pltpu.PrefetchScalarGridSpec(
        num_scalar_prefetch=5,
        grid=(_NSTEPS,),
        in_specs=[xspec, lspec, lspec, lspec, lspec],
        out_specs=[ospec] + [lspec] * 8,
        scratch_shapes=[
            pltpu.VMEM(xblock, x.dtype),
            pltpu.VMEM(xblock, x.dtype),
        ],
    )
    lab_sds = jax.ShapeDtypeStruct(lab_shape, cls_labels.dtype)
    outs = pl.pallas_call(
        _mix_body,
        grid_spec=grid_spec,
        out_shape=[jax.ShapeDtypeStruct(x.shape, x.dtype)] + [lab_sds] * 8,
    )(pidx, lidx, oidx, flags, cstep, x, cls_labels, reg_labels, cls_masks, reg_masks)
    (x_mixed, cls1, cls2, reg1, reg2, cm1, cm2, rm1, rm2) = outs
    return (x_mixed, cls1, cls2, reg1, reg2, cm1, cm2, rm1, rm2, coeffs, pidx)


# speedup vs baseline: 3.5023x; 1.3935x over previous
import jax
import jax.numpy as jnp
import numpy as np
from jax import lax
from jax.experimental import pallas as pl
from jax.experimental.pallas import tpu as pltpu

_BS = 64
_K = 4

_PERM_NP = np.array([
    17, 27, 42, 32, 1, 3, 58, 51, 40, 28, 52, 19, 9, 33, 11, 45, 31, 5, 15,
    39, 50, 47, 20, 0, 46, 14, 49, 44, 38, 61, 2, 54, 36, 35, 62, 63, 21, 59,
    30, 43, 22, 18, 24, 26, 53, 12, 16, 6, 7, 57, 55, 48, 13, 37, 60, 10, 29,
    34, 25, 56, 4, 41, 23, 8], dtype=np.int32)
_COEFFS_NP = np.array([
    1037351011, 1061372630, 1057324213, 1056363742, 1063086089, 1057807661,
    1040386029, 1065181069, 1058026594, 1020609760, 1065181398, 1059614811,
    1061364246, 1065181069, 1062492239, 978165541, 1024555604, 1063824199,
    1035934354, 1059732161, 1064790172, 1063985662, 1057562209, 1061392501,
    1064987886, 1019645466, 1054168645, 1053640420, 1065263794, 1063244784,
    1046450749, 1009553876, 999950345, 1035548033, 1060487295, 1065236971,
    1037171929, 1025682675, 1009050473, 1062548471, 1050146486, 1065145350,
    1022592052, 1064836962, 1062864128, 1050453788, 1050563139, 1051970733,
    1062604949, 1043085377, 1044443892, 1065333697, 1033373725, 1048891341,
    1065037049, 1054181325, 1038811005, 997617312, 1048404752, 1015544083,
    1064025317, 1049906982, 1060546158, 1018825991],
    dtype=np.uint32).view(np.float32)


def _ring_schedule(perm, coeffs, k):
    n = len(perm)
    seen = np.zeros(n, dtype=bool)
    load_idx, out_idx, has_out, c_step = [], [], [], []
    for s in range(n):
        if seen[s]:
            continue
        cyc = []
        j = s
        while not seen[j]:
            seen[j] = True
            cyc.append(j)
            j = int(perm[j])
        ln = len(cyc)
        load_idx.append(cyc[0])
        out_idx.append(0)
        has_out.append(0)
        c_step.append(0.0)
        for t in range(1, ln + 1):
            load_idx.append(cyc[t % ln])
            out_idx.append(cyc[t - 1])
            has_out.append(1)
            c_step.append(float(coeffs[cyc[t - 1]]))
    nsteps = len(load_idx)
    wait_out = np.zeros(nsteps, dtype=np.int32)
    pending = [False] * k
    for g in range(nsteps):
        if has_out[g]:
            if pending[g % k]:
                wait_out[g] = 1
            pending[g % k] = True
    pad = np.zeros(k, dtype=np.int32)
    return (
        np.concatenate([np.asarray(load_idx, np.int32), pad]),
        np.asarray(out_idx, np.int32),
        np.asarray(has_out, np.int32),
        wait_out,
        np.asarray(c_step, np.float32),
        np.asarray(pending, np.bool_),
        nsteps,
    )


(_LOAD_NP, _OUT_NP, _HASOUT_NP, _WAITOUT_NP, _CSTEP_NP, _DRAIN_NP,
 _NSTEPS) = _ring_schedule(_PERM_NP, _COEFFS_NP, _K)


def _mix_body(lidx_ref, oidx_ref, hout_ref, wout_ref, c_ref, pidx_ref,
              x_hbm, cls_ref, reg_ref, cm_ref, rm_ref,
              out_hbm, o_cls1, o_cls2, o_reg1, o_reg2, o_cm1, o_cm2,
              o_rm1, o_rm2, inbuf, outbuf, in_sems, out_sems):
    for h in range(_K - 1):
        pltpu.make_async_copy(
            x_hbm.at[lidx_ref[h]], inbuf.at[h], in_sems.at[h]).start()

    o_cls1[...] = cls_ref[...]
    o_reg1[...] = reg_ref[...]
    o_cm1[...] = cm_ref[...]
    o_rm1[...] = rm_ref[...]

    def gather_row(j, _):
        p = pidx_ref[j]
        o_cls2[pl.ds(j, 1), :] = cls_ref[pl.ds(p, 1), :]
        o_reg2[pl.ds(j, 1), :] = reg_ref[pl.ds(p, 1), :]
        o_cm2[pl.ds(j, 1), :] = cm_ref[pl.ds(p, 1), :]
        o_rm2[pl.ds(j, 1), :] = rm_ref[pl.ds(p, 1), :]
        return 0

    lax.fori_loop(0, _BS, gather_row, 0)

    def step(g, _):
        slot = lax.rem(g, _K)
        prev_slot = lax.rem(g + (_K - 1), _K)
        pltpu.make_async_copy(
            x_hbm.at[lidx_ref[g]], inbuf.at[slot], in_sems.at[slot]).wait()

        @pl.when(hout_ref[g] == 1)
        def _():
            @pl.when(wout_ref[g] == 1)
            def _():
                pltpu.make_async_copy(
                    outbuf.at[slot], out_hbm.at[oidx_ref[g]],
                    out_sems.at[slot]).wait()

            c = c_ref[g]
            outbuf[slot] = c * inbuf[prev_slot] + (1.0 - c) * inbuf[slot]
            pltpu.make_async_copy(
                outbuf.at[slot], out_hbm.at[oidx_ref[g]],
                out_sems.at[slot]).start()

        h = g + _K - 1

        @pl.when(h < _NSTEPS)
        def _():
            pltpu.make_async_copy(
                x_hbm.at[lidx_ref[h]], inbuf.at[prev_slot],
                in_sems.at[prev_slot]).start()

        return 0

    lax.fori_loop(0, _NSTEPS, step, 0)

    for s in range(_K):
        if _DRAIN_NP[s]:
            pltpu.make_async_copy(
                outbuf.at[s], out_hbm.at[0], out_sems.at[s]).wait()


def kernel(x, cls_labels, reg_labels, cls_masks, reg_masks):
    lab_shape = cls_labels.shape
    row = (x.shape[2], x.shape[3])
    x3 = x.reshape(x.shape[0], *row)

    lidx = jnp.asarray(_LOAD_NP)
    oidx = jnp.asarray(_OUT_NP)
    hout = jnp.asarray(_HASOUT_NP)
    wout = jnp.asarray(_WAITOUT_NP)
    cstep = jnp.asarray(_CSTEP_NP)
    pidx = jnp.asarray(_PERM_NP, dtype=jnp.int32)
    coeffs = jnp.asarray(_COEFFS_NP, dtype=jnp.float32)

    smem = pl.BlockSpec(memory_space=pltpu.SMEM)
    anys = pl.BlockSpec(memory_space=pl.ANY)
    vmem = pl.BlockSpec(memory_space=pltpu.VMEM)

    lab_sds = jax.ShapeDtypeStruct(lab_shape, cls_labels.dtype)
    outs = pl.pallas_call(
        _mix_body,
        in_specs=[smem] * 6 + [anys] + [vmem] * 4,
        out_specs=[anys] + [vmem] * 8,
        out_shape=[jax.ShapeDtypeStruct(x3.shape, x.dtype)] + [lab_sds] * 8,
        scratch_shapes=[
            pltpu.VMEM((_K,) + row, x.dtype),
            pltpu.VMEM((_K,) + row, x.dtype),
            pltpu.SemaphoreType.DMA((_K,)),
            pltpu.SemaphoreType.DMA((_K,)),
        ],
    )(lidx, oidx, hout, wout, cstep, pidx,
      x3, cls_labels, reg_labels, cls_masks, reg_masks)
    (xm, cls1, cls2, reg1, reg2, cm1, cm2, rm1, rm2) = outs
    return (xm.reshape(x.shape), cls1, cls2, reg1, reg2, cm1, cm2, rm1, rm2,
            coeffs, pidx)


# speedup vs baseline: 3.9109x; 1.1167x over previous
import jax
import jax.numpy as jnp
import numpy as np
from jax import lax
from jax.experimental import pallas as pl
from jax.experimental.pallas import tpu as pltpu

_BS = 64
_K = 8

_PERM_NP = np.array([
    17, 27, 42, 32, 1, 3, 58, 51, 40, 28, 52, 19, 9, 33, 11, 45, 31, 5, 15,
    39, 50, 47, 20, 0, 46, 14, 49, 44, 38, 61, 2, 54, 36, 35, 62, 63, 21, 59,
    30, 43, 22, 18, 24, 26, 53, 12, 16, 6, 7, 57, 55, 48, 13, 37, 60, 10, 29,
    34, 25, 56, 4, 41, 23, 8], dtype=np.int32)
_COEFFS_NP = np.array([
    1037351011, 1061372630, 1057324213, 1056363742, 1063086089, 1057807661,
    1040386029, 1065181069, 1058026594, 1020609760, 1065181398, 1059614811,
    1061364246, 1065181069, 1062492239, 978165541, 1024555604, 1063824199,
    1035934354, 1059732161, 1064790172, 1063985662, 1057562209, 1061392501,
    1064987886, 1019645466, 1054168645, 1053640420, 1065263794, 1063244784,
    1046450749, 1009553876, 999950345, 1035548033, 1060487295, 1065236971,
    1037171929, 1025682675, 1009050473, 1062548471, 1050146486, 1065145350,
    1022592052, 1064836962, 1062864128, 1050453788, 1050563139, 1051970733,
    1062604949, 1043085377, 1044443892, 1065333697, 1033373725, 1048891341,
    1065037049, 1054181325, 1038811005, 997617312, 1048404752, 1015544083,
    1064025317, 1049906982, 1060546158, 1018825991],
    dtype=np.uint32).view(np.float32)


def _ring_schedule(perm, coeffs, k):
    n = len(perm)
    seen = np.zeros(n, dtype=bool)
    load_idx, out_idx, has_out, c_step = [], [], [], []
    for s in range(n):
        if seen[s]:
            continue
        cyc = []
        j = s
        while not seen[j]:
            seen[j] = True
            cyc.append(j)
            j = int(perm[j])
        ln = len(cyc)
        load_idx.append(cyc[0])
        out_idx.append(0)
        has_out.append(0)
        c_step.append(0.0)
        for t in range(1, ln + 1):
            load_idx.append(cyc[t % ln])
            out_idx.append(cyc[t - 1])
            has_out.append(1)
            c_step.append(float(coeffs[cyc[t - 1]]))
    nsteps = len(load_idx)
    wait_out = np.zeros(nsteps, dtype=np.int32)
    pending = [False] * k
    for g in range(nsteps):
        if has_out[g]:
            if pending[g % k]:
                wait_out[g] = 1
            pending[g % k] = True
    pad = np.zeros(k, dtype=np.int32)
    return (
        np.concatenate([np.asarray(load_idx, np.int32), pad]),
        np.asarray(out_idx, np.int32),
        np.asarray(has_out, np.int32),
        wait_out,
        np.asarray(c_step, np.float32),
        np.asarray(pending, np.bool_),
        nsteps,
    )


(_LOAD_NP, _OUT_NP, _HASOUT_NP, _WAITOUT_NP, _CSTEP_NP, _DRAIN_NP,
 _NSTEPS) = _ring_schedule(_PERM_NP, _COEFFS_NP, _K)


def _mix_body(lidx_ref, oidx_ref, hout_ref, wout_ref, c_ref, pidx_ref,
              x_hbm, cls_ref, reg_ref, cm_ref, rm_ref,
              out_hbm, o_cls1, o_cls2, o_reg1, o_reg2, o_cm1, o_cm2,
              o_rm1, o_rm2, inbuf, outbuf, in_sems, out_sems):
    for h in range(_K - 1):
        pltpu.make_async_copy(
            x_hbm.at[lidx_ref[h]], inbuf.at[h], in_sems.at[h]).start()

    o_cls1[...] = cls_ref[...]
    o_reg1[...] = reg_ref[...]
    o_cm1[...] = cm_ref[...]
    o_rm1[...] = rm_ref[...]

    def gather_row(j, _):
        p = pidx_ref[j]
        o_cls2[pl.ds(j, 1), :] = cls_ref[pl.ds(p, 1), :]
        o_reg2[pl.ds(j, 1), :] = reg_ref[pl.ds(p, 1), :]
        o_cm2[pl.ds(j, 1), :] = cm_ref[pl.ds(p, 1), :]
        o_rm2[pl.ds(j, 1), :] = rm_ref[pl.ds(p, 1), :]
        return 0

    lax.fori_loop(0, _BS, gather_row, 0)

    def step(g, _):
        slot = lax.rem(g, _K)
        prev_slot = lax.rem(g + (_K - 1), _K)
        pltpu.make_async_copy(
            x_hbm.at[lidx_ref[g]], inbuf.at[slot], in_sems.at[slot]).wait()

        @pl.when(hout_ref[g] == 1)
        def _():
            @pl.when(wout_ref[g] == 1)
            def _():
                pltpu.make_async_copy(
                    outbuf.at[slot], out_hbm.at[oidx_ref[g]],
                    out_sems.at[slot]).wait()

            c = c_ref[g]
            outbuf[slot] = c * inbuf[prev_slot] + (1.0 - c) * inbuf[slot]
            pltpu.make_async_copy(
                outbuf.at[slot], out_hbm.at[oidx_ref[g]],
                out_sems.at[slot]).start()

        h = g + _K - 1

        @pl.when(h < _NSTEPS)
        def _():
            pltpu.make_async_copy(
                x_hbm.at[lidx_ref[h]], inbuf.at[prev_slot],
                in_sems.at[prev_slot]).start()

        return 0

    lax.fori_loop(0, _NSTEPS, step, 0)

    for s in range(_K):
        if _DRAIN_NP[s]:
            pltpu.make_async_copy(
                outbuf.at[s], out_hbm.at[0], out_sems.at[s]).wait()


def kernel(x, cls_labels, reg_labels, cls_masks, reg_masks):
    lab_shape = cls_labels.shape
    row = (x.shape[2], x.shape[3])
    x3 = x.reshape(x.shape[0], *row)

    lidx = jnp.asarray(_LOAD_NP)
    oidx = jnp.asarray(_OUT_NP)
    hout = jnp.asarray(_HASOUT_NP)
    wout = jnp.asarray(_WAITOUT_NP)
    cstep = jnp.asarray(_CSTEP_NP)
    pidx = jnp.asarray(_PERM_NP, dtype=jnp.int32)
    coeffs = jnp.asarray(_COEFFS_NP, dtype=jnp.float32)

    smem = pl.BlockSpec(memory_space=pltpu.SMEM)
    anys = pl.BlockSpec(memory_space=pl.ANY)
    vmem = pl.BlockSpec(memory_space=pltpu.VMEM)

    lab_sds = jax.ShapeDtypeStruct(lab_shape, cls_labels.dtype)
    outs = pl.pallas_call(
        _mix_body,
        in_specs=[smem] * 6 + [anys] + [vmem] * 4,
        out_specs=[anys] + [vmem] * 8,
        out_shape=[jax.ShapeDtypeStruct(x3.shape, x.dtype)] + [lab_sds] * 8,
        scratch_shapes=[
            pltpu.VMEM((_K,) + row, x.dtype),
            pltpu.VMEM((_K,) + row, x.dtype),
            pltpu.SemaphoreType.DMA((_K,)),
            pltpu.SemaphoreType.DMA((_K,)),
        ],
    )(lidx, oidx, hout, wout, cstep, pidx,
      x3, cls_labels, reg_labels, cls_masks, reg_masks)
    (xm, cls1, cls2, reg1, reg2, cm1, cm2, rm1, rm2) = outs
    return (xm.reshape(x.shape), cls1, cls2, reg1, reg2, cm1, cm2, rm1, rm2,
            coeffs, pidx)
